# baseline (speedup 1.0000x reference)
import functools

import jax
import jax.numpy as jnp
from jax.experimental import pallas as pl
from jax.experimental.pallas import tpu as pltpu

_NUM_EXPERTS = 16
_TOP_K = 2
_BLK = 1024


def _gate_kernel(x_hbm, w_ref, s_ref, i_ref, xbuf, sems):
    step = pl.program_id(0)
    n = pl.num_programs(0)
    slot = jax.lax.rem(step, 2)
    nxt = jax.lax.rem(step + 1, 2)

    @pl.when(step == 0)
    def _():
        pltpu.make_async_copy(
            x_hbm.at[pl.ds(0, _BLK)], xbuf.at[0], sems.at[0]
        ).start()

    @pl.when(step + 1 < n)
    def _():
        pltpu.make_async_copy(
            x_hbm.at[pl.ds((step + 1) * _BLK, _BLK)], xbuf.at[nxt], sems.at[nxt]
        ).start()

    pltpu.make_async_copy(
        x_hbm.at[pl.ds(step * _BLK, _BLK)], xbuf.at[slot], sems.at[slot]
    ).wait()

    lt = jax.lax.dot_general(
        w_ref[...], xbuf[slot], (((1,), (1,)), ((), ())),
        preferred_element_type=jnp.float32,
    )
    m = jnp.max(lt, axis=0, keepdims=True)
    row = jax.lax.broadcasted_iota(jnp.int32, lt.shape, 0)
    i1 = jnp.min(jnp.where(lt == m, row, _NUM_EXPERTS), axis=0, keepdims=True)
    masked = jnp.where(row == i1, -jnp.inf, lt)
    l2 = jnp.max(masked, axis=0, keepdims=True)
    i2 = jnp.min(jnp.where(masked == l2, row, _NUM_EXPERTS), axis=0, keepdims=True)
    z = jnp.sum(jnp.exp(lt - m), axis=0, keepdims=True)

    v1 = 1.0 / z
    t = jnp.exp(jnp.exp(l2 - m) / z - v1)
    d = 1.0 + t
    s_ref[...] = jnp.concatenate([1.0 / d, t / d], axis=0).T
    i_ref[...] = jnp.concatenate([i1, i2], axis=0).T


@functools.partial(jax.jit, static_argnums=())
def kernel(x, W_gate):
    b, s, h = x.shape
    rows = b * s
    x_flat = x.reshape(rows, h)
    grid = (rows // _BLK,)
    scores, idx = pl.pallas_call(
        _gate_kernel,
        grid=grid,
        in_specs=[
            pl.BlockSpec(memory_space=pl.ANY),
            pl.BlockSpec((_NUM_EXPERTS, h), lambda i: (0, 0)),
        ],
        out_specs=[
            pl.BlockSpec((_BLK, _TOP_K), lambda i: (i, 0)),
            pl.BlockSpec((_BLK, _TOP_K), lambda i: (i, 0)),
        ],
        out_shape=[
            jax.ShapeDtypeStruct((rows, _TOP_K), jnp.float32),
            jax.ShapeDtypeStruct((rows, _TOP_K), jnp.int32),
        ],
        scratch_shapes=[
            pltpu.VMEM((2, _BLK, 2048), jnp.float32),
            pltpu.SemaphoreType.DMA((2,)),
        ],
        compiler_params=pltpu.CompilerParams(
            dimension_semantics=("arbitrary",),
        ),
    )(x_flat, W_gate)
    aux_loss = jnp.array(0.0, dtype=jnp.float32)
    return (scores, idx, aux_loss)
